# Initial kernel scaffold; baseline (speedup 1.0000x reference)
import jax
import jax.numpy as jnp
from jax.experimental import pallas as pl


def kernel(x, table_0, table_1, table_2, table_3, table_4, table_5, table_6, table_7, table_8, table_9, table_10, table_11, table_12, table_13, table_14, table_15):
    raise NotImplementedError("write your pallas kernel here")


# speedup vs baseline: 90.3083x; 90.3083x over previous
import math

import jax
import jax.numpy as jnp
from jax import lax
from jax.experimental import pallas as pl
from jax.experimental.pallas import tpu as pltpu
from jax.experimental.pallas import tpu_sc as plsc

DIM = 3
N_LEVELS = 16
N_FEATS = 2
LOG2_HASHMAP = 19
BASE_RES = 16
FINEST_RES = 512
N_POINTS = 524288
PRIME1 = 2654435761
PRIME2 = 805459861
OUT_F = 2 * N_LEVELS
ROW = 16
EPR = ROW // N_FEATS


def _levels():
    b = math.exp((math.log(FINEST_RES) - math.log(BASE_RES)) / (N_LEVELS - 1))
    out = []
    for i in range(N_LEVELS):
        res = math.floor(BASE_RES * (b ** i))
        hsize = min(res ** DIM, 2 ** LOG2_HASHMAP)
        out.append((res, hsize))
    return out


LEVELS = _levels()

NC = 2
NS = 16
LANES = 16
NW = NC * NS
PTS_PER_W = N_POINTS // NW
P = 256
CHUNKS = PTS_PER_W // P
G = P // LANES
K = 8 * P


def _sc_body(xT_hbm, t0, t1, t2, t3, t4, t5, t6, t7, t8, t9, t10, t11, t12,
             t13, t14, t15, out_hbm, x_v, idx_v, sub_v, w_v, rows_v, out_v,
             sem_a, sem_b):
    tables = [t0, t1, t2, t3, t4, t5, t6, t7, t8, t9, t10, t11, t12, t13,
              t14, t15]
    sems = [sem_a, sem_b]
    c = lax.axis_index("c")
    s = lax.axis_index("s")
    wid = s * NC + c
    lane = lax.iota(jnp.int32, LANES)

    def chunk_body(ci, carry):
        base = wid * PTS_PER_W + ci * P

        def idx_level(l, b):
            res, hsize = LEVELS[l]
            resf = jnp.float32(res)
            pow2 = (hsize & (hsize - 1)) == 0

            def idx_body(g, _, hsize=hsize, pow2=pow2, resf=resf, b=b):
                p0 = g * LANES
                xs0 = x_v[0, pl.ds(p0, LANES)] * resf
                xs1 = x_v[1, pl.ds(p0, LANES)] * resf
                xs2 = x_v[2, pl.ds(p0, LANES)] * resf
                i0 = xs0.astype(jnp.int32)
                i1 = xs1.astype(jnp.int32)
                i2 = xs2.astype(jnp.int32)
                f0 = xs0 - i0.astype(jnp.float32)
                f1 = xs1 - i1.astype(jnp.float32)
                f2 = xs2 - i2.astype(jnp.float32)
                u0a = i0.astype(jnp.uint32)
                u1a = i1.astype(jnp.uint32) * jnp.uint32(PRIME1)
                u2a = i2.astype(jnp.uint32) * jnp.uint32(PRIME2)
                u0b = u0a + jnp.uint32(1)
                u1b = u1a + jnp.uint32(PRIME1)
                u2b = u2a + jnp.uint32(PRIME2)
                g0 = 1.0 - f0
                g1 = 1.0 - f1
                g2 = 1.0 - f2
                for cidx in range(8):
                    hx = u0b if (cidx & 1) else u0a
                    hy = u1b if (cidx & 2) else u1a
                    hz = u2b if (cidx & 4) else u2a
                    h = hx ^ hy ^ hz
                    if pow2:
                        h = h & jnp.uint32(hsize - 1)
                    else:
                        h = h % jnp.uint32(hsize)
                    h = h.astype(jnp.int32)
                    idx_v[b, pl.ds(cidx * P + p0, LANES)] = (
                        lax.shift_right_logical(h, 3))
                    sub_v[b, pl.ds(cidx * P + p0, LANES)] = (h & 7) * N_FEATS
                    wx = f0 if (cidx & 1) else g0
                    wy = f1 if (cidx & 2) else g1
                    wz = f2 if (cidx & 4) else g2
                    w_v[b, pl.ds(cidx * P + p0, LANES)] = wx * wy * wz
                return 0

            lax.fori_loop(0, G, idx_body, 0)
            return pltpu.async_copy(tables[l].at[idx_v.at[b]],
                                    rows_v.at[b], sems[b])

        def acc_level(l, b):
            def acc_body(g, _, l=l, b=b):
                p0 = g * LANES
                pos = p0 + lane
                bvec = jnp.full((LANES,), b, jnp.int32)
                acc0 = jnp.zeros((LANES,), jnp.float32)
                acc1 = jnp.zeros((LANES,), jnp.float32)
                for cidx in range(8):
                    rpos = pos + cidx * P
                    sub = sub_v[b, pl.ds(cidx * P + p0, LANES)]
                    r0 = plsc.load_gather(rows_v, [bvec, rpos, sub])
                    r1 = plsc.load_gather(rows_v, [bvec, rpos, sub + 1])
                    w = w_v[b, pl.ds(cidx * P + p0, LANES)]
                    acc0 = acc0 + w * r0
                    acc1 = acc1 + w * r1
                plsc.store_scatter(
                    out_v, [pos, jnp.full((LANES,), 2 * l, jnp.int32)], acc0)
                plsc.store_scatter(
                    out_v, [pos, jnp.full((LANES,), 2 * l + 1, jnp.int32)],
                    acc1)
                return 0

            lax.fori_loop(0, G, acc_body, 0)

        pltpu.sync_copy(xT_hbm.at[:, pl.ds(base, P)], x_v)
        cps = [None, None]
        cps[0] = idx_level(0, 0)
        for l in range(N_LEVELS):
            b = l % 2
            if l + 1 < N_LEVELS:
                cps[1 - b] = idx_level(l + 1, 1 - b)
            cps[b].wait()
            acc_level(l, b)
        pltpu.sync_copy(out_v, out_hbm.at[pl.ds(base, P), :])
        return carry

    lax.fori_loop(0, CHUNKS, chunk_body, 0)


def _as_rows(table):
    hsize = table.shape[0]
    pad = (-hsize) % EPR
    if pad:
        table = jnp.concatenate(
            [table, jnp.zeros((pad, N_FEATS), table.dtype)], axis=0)
    return table.reshape(-1, ROW)


def kernel(x, table_0, table_1, table_2, table_3, table_4, table_5, table_6,
           table_7, table_8, table_9, table_10, table_11, table_12, table_13,
           table_14, table_15):
    xT = x.T
    tables = [table_0, table_1, table_2, table_3, table_4, table_5, table_6,
              table_7, table_8, table_9, table_10, table_11, table_12,
              table_13, table_14, table_15]
    trows = [_as_rows(t) for t in tables]
    mesh = plsc.VectorSubcoreMesh(core_axis_name="c", subcore_axis_name="s")
    f = pl.kernel(
        _sc_body,
        out_type=jax.ShapeDtypeStruct((N_POINTS, OUT_F), jnp.float32),
        mesh=mesh,
        compiler_params=pltpu.CompilerParams(
            needs_layout_passes=False, use_tc_tiling_on_sc=False),
        scratch_types=[
            pltpu.VMEM((3, P), jnp.float32),
            pltpu.VMEM((2, K), jnp.int32),
            pltpu.VMEM((2, K), jnp.int32),
            pltpu.VMEM((2, K), jnp.float32),
            pltpu.VMEM((2, K, ROW), jnp.float32),
            pltpu.VMEM((P, OUT_F), jnp.float32),
            pltpu.SemaphoreType.DMA,
            pltpu.SemaphoreType.DMA,
        ],
    )
    return f(xT, *trows)


# speedup vs baseline: 157.3509x; 1.7424x over previous
import math

import jax
import jax.numpy as jnp
from jax import lax
from jax.experimental import pallas as pl
from jax.experimental.pallas import tpu as pltpu
from jax.experimental.pallas import tpu_sc as plsc

DIM = 3
N_LEVELS = 16
N_FEATS = 2
LOG2_HASHMAP = 19
BASE_RES = 16
FINEST_RES = 512
N_POINTS = 524288
PRIME1 = 2654435761
PRIME2 = 805459861
OUT_F = 2 * N_LEVELS
ROW = 16
BLK = 128


def _levels():
    b = math.exp((math.log(FINEST_RES) - math.log(BASE_RES)) / (N_LEVELS - 1))
    out = []
    for i in range(N_LEVELS):
        res = math.floor(BASE_RES * (b ** i))
        hsize = min(res ** DIM, 2 ** LOG2_HASHMAP)
        out.append((res, hsize))
    return out


LEVELS = _levels()

NC = 2
NS = 16
LANES = 16
NW = NC * NS
PTS_PER_W = N_POINTS // NW
P = 256
CHUNKS = PTS_PER_W // P
G = P // LANES
K = 8 * P

PAD_H = [max(2048, 1 << (h - 1).bit_length()) for _, h in LEVELS]
BLOCKS = [h // BLK for h in PAD_H]
SHARE = [b // NS for b in BLOCKS]
ROWS_L = [h // 8 for h in PAD_H]
LEVEL_OFF = [0]
for r in ROWS_L:
    LEVEL_OFF.append(LEVEL_OFF[-1] + r)
TOT_ROWS = LEVEL_OFF[-1]

NAT_CH = 32


def _sc_body(xT_hbm, t0, t1, t2, t3, t4, t5, t6, t7, t8, t9, t10, t11, t12,
             t13, t14, t15, out_hbm, x_v, idx_v, sub_v, w_v, rows_v,
             out_v, nat_v, pk_v, pk_hbm, sem_a, sem_b):
    tables = [t0, t1, t2, t3, t4, t5, t6, t7, t8, t9, t10, t11, t12, t13,
              t14, t15]
    sems = [sem_a, sem_b]
    c = lax.axis_index("c")
    s = lax.axis_index("s")
    wid = s * NC + c
    lane = lax.iota(jnp.int32, LANES)
    sc_base = c * TOT_ROWS

    for l in range(N_LEVELS):
        sh = SHARE[l]
        ch = min(NAT_CH, sh)
        n_ch = sh // ch
        ng = ch * 8

        def rep_chunk(j, _, l=l, sh=sh, ch=ch, ng=ng):
            b0 = s * sh + j * ch
            pltpu.sync_copy(tables[l].at[pl.ds(b0, ch), :, :],
                            nat_v.at[pl.ds(0, ch), :, :])

            def rep_grp(g, _):
                e0 = g * LANES
                blk = lax.div(e0, BLK)
                eo = lax.rem(e0, BLK)
                f0 = nat_v[blk, 0, pl.ds(eo, LANES)]
                f1 = nat_v[blk, 1, pl.ds(eo, LANES)]
                pos = 2 * (e0 + lane)
                r = lax.shift_right_logical(pos, 4)
                sub = pos & 15
                plsc.store_scatter(pk_v, [r, sub], f0)
                plsc.store_scatter(pk_v, [r, sub + 1], f1)
                return 0

            lax.fori_loop(0, ng, rep_grp, 0)
            row0 = sc_base + LEVEL_OFF[l] + (b0 * (BLK // 8))
            pltpu.sync_copy(pk_v.at[pl.ds(0, ch * (BLK // 8)), :],
                            pk_hbm.at[pl.ds(row0, ch * (BLK // 8)), :])
            return 0

        lax.fori_loop(0, n_ch, rep_chunk, 0)

    plsc.subcore_barrier()

    def chunk_body(ci, carry):
        base = wid * PTS_PER_W + ci * P

        def idx_level(l, b):
            res, hsize = LEVELS[l]
            resf = jnp.float32(res)
            pow2 = (hsize & (hsize - 1)) == 0
            base_l = sc_base + LEVEL_OFF[l]

            def idx_body(g, _, hsize=hsize, pow2=pow2, resf=resf, b=b,
                         base_l=base_l):
                p0 = g * LANES
                xs0 = x_v[0, pl.ds(p0, LANES)] * resf
                xs1 = x_v[1, pl.ds(p0, LANES)] * resf
                xs2 = x_v[2, pl.ds(p0, LANES)] * resf
                i0 = xs0.astype(jnp.int32)
                i1 = xs1.astype(jnp.int32)
                i2 = xs2.astype(jnp.int32)
                f0 = xs0 - i0.astype(jnp.float32)
                f1 = xs1 - i1.astype(jnp.float32)
                f2 = xs2 - i2.astype(jnp.float32)
                u0a = i0.astype(jnp.uint32)
                u1a = i1.astype(jnp.uint32) * jnp.uint32(PRIME1)
                u2a = i2.astype(jnp.uint32) * jnp.uint32(PRIME2)
                u0b = u0a + jnp.uint32(1)
                u1b = u1a + jnp.uint32(PRIME1)
                u2b = u2a + jnp.uint32(PRIME2)
                g0 = 1.0 - f0
                g1 = 1.0 - f1
                g2 = 1.0 - f2
                for cidx in range(8):
                    hx = u0b if (cidx & 1) else u0a
                    hy = u1b if (cidx & 2) else u1a
                    hz = u2b if (cidx & 4) else u2a
                    h = hx ^ hy ^ hz
                    if pow2:
                        h = h & jnp.uint32(hsize - 1)
                    else:
                        h = h % jnp.uint32(hsize)
                    h = h.astype(jnp.int32)
                    idx_v[b, pl.ds(cidx * P + p0, LANES)] = (
                        base_l + lax.shift_right_logical(h, 3))
                    sub_v[b, pl.ds(cidx * P + p0, LANES)] = (h & 7) * N_FEATS
                    wx = f0 if (cidx & 1) else g0
                    wy = f1 if (cidx & 2) else g1
                    wz = f2 if (cidx & 4) else g2
                    w_v[b, pl.ds(cidx * P + p0, LANES)] = wx * wy * wz
                return 0

            lax.fori_loop(0, G, idx_body, 0)
            return pltpu.async_copy(pk_hbm.at[idx_v.at[b]],
                                    rows_v.at[b], sems[b])

        def acc_level(l, b):
            def acc_body(g, _, l=l, b=b):
                p0 = g * LANES
                pos = p0 + lane
                bvec = jnp.full((LANES,), b, jnp.int32)
                acc0 = jnp.zeros((LANES,), jnp.float32)
                acc1 = jnp.zeros((LANES,), jnp.float32)
                for cidx in range(8):
                    rpos = pos + cidx * P
                    sub = sub_v[b, pl.ds(cidx * P + p0, LANES)]
                    r0 = plsc.load_gather(rows_v, [bvec, rpos, sub])
                    r1 = plsc.load_gather(rows_v, [bvec, rpos, sub + 1])
                    w = w_v[b, pl.ds(cidx * P + p0, LANES)]
                    acc0 = acc0 + w * r0
                    acc1 = acc1 + w * r1
                plsc.store_scatter(
                    out_v, [pos, jnp.full((LANES,), 2 * l, jnp.int32)], acc0)
                plsc.store_scatter(
                    out_v, [pos, jnp.full((LANES,), 2 * l + 1, jnp.int32)],
                    acc1)
                return 0

            lax.fori_loop(0, G, acc_body, 0)

        pltpu.sync_copy(xT_hbm.at[:, pl.ds(base, P)], x_v)
        cps = [None, None]
        cps[0] = idx_level(0, 0)
        for l in range(N_LEVELS):
            b = l % 2
            if l + 1 < N_LEVELS:
                cps[1 - b] = idx_level(l + 1, 1 - b)
            cps[b].wait()
            acc_level(l, b)
        pltpu.sync_copy(out_v, out_hbm.at[pl.ds(base, P), :])
        return carry

    lax.fori_loop(0, CHUNKS, chunk_body, 0)


def _native_view(table, hpad):
    hsize = table.shape[0]
    if hpad != hsize:
        table = jnp.concatenate(
            [table, jnp.zeros((hpad - hsize, N_FEATS), table.dtype)], axis=0)
    return table.reshape(hpad // BLK, BLK, N_FEATS).transpose(0, 2, 1)


def kernel(x, table_0, table_1, table_2, table_3, table_4, table_5, table_6,
           table_7, table_8, table_9, table_10, table_11, table_12, table_13,
           table_14, table_15):
    xT = x.T
    tables = [table_0, table_1, table_2, table_3, table_4, table_5, table_6,
              table_7, table_8, table_9, table_10, table_11, table_12,
              table_13, table_14, table_15]
    tviews = [_native_view(t, hp) for t, hp in zip(tables, PAD_H)]
    mesh = plsc.VectorSubcoreMesh(core_axis_name="c", subcore_axis_name="s")
    f = pl.kernel(
        _sc_body,
        out_type=jax.ShapeDtypeStruct((N_POINTS, OUT_F), jnp.float32),
        mesh=mesh,
        compiler_params=pltpu.CompilerParams(
            needs_layout_passes=False, use_tc_tiling_on_sc=False),
        scratch_types=[
            pltpu.VMEM((3, P), jnp.float32),
            pltpu.VMEM((2, K), jnp.int32),
            pltpu.VMEM((2, K), jnp.int32),
            pltpu.VMEM((2, K), jnp.float32),
            pltpu.VMEM((2, K, ROW), jnp.float32),
            pltpu.VMEM((P, OUT_F), jnp.float32),
            pltpu.VMEM((NAT_CH, N_FEATS, BLK), jnp.float32),
            pltpu.VMEM((NAT_CH * (BLK // 8), ROW), jnp.float32),
            pltpu.HBM((NC * TOT_ROWS, ROW), jnp.float32),
            pltpu.SemaphoreType.DMA,
            pltpu.SemaphoreType.DMA,
        ],
    )
    return f(xT, *tviews)


# speedup vs baseline: 199.9031x; 1.2704x over previous
import math

import jax
import jax.numpy as jnp
from jax import lax
from jax.experimental import pallas as pl
from jax.experimental.pallas import tpu as pltpu
from jax.experimental.pallas import tpu_sc as plsc

DIM = 3
N_LEVELS = 16
N_FEATS = 2
LOG2_HASHMAP = 19
BASE_RES = 16
FINEST_RES = 512
N_POINTS = 524288
PRIME1 = 2654435761
PRIME2 = 805459861
OUT_F = 2 * N_LEVELS
ROW = 16
BLK = 128


def _levels():
    b = math.exp((math.log(FINEST_RES) - math.log(BASE_RES)) / (N_LEVELS - 1))
    out = []
    for i in range(N_LEVELS):
        res = math.floor(BASE_RES * (b ** i))
        hsize = min(res ** DIM, 2 ** LOG2_HASHMAP)
        out.append((res, hsize))
    return out


LEVELS = _levels()

NC = 2
NS = 16
LANES = 16
NW = NC * NS
PTS_PER_W = N_POINTS // NW
P = 256
CHUNKS = PTS_PER_W // P
G = P // LANES
K = 8 * P

PAD_H = [max(2048, 1 << (h - 1).bit_length()) for _, h in LEVELS]
BLOCKS = [h // BLK for h in PAD_H]
SHARE = [b // NS for b in BLOCKS]
ROWS_L = [h // 8 for h in PAD_H]
N_SPM = 6
LEVEL_OFF = []
off_s, off_h = 0, 0
for l, r in enumerate(ROWS_L):
    if l < N_SPM:
        LEVEL_OFF.append(off_s)
        off_s += r
    else:
        LEVEL_OFF.append(off_h)
        off_h += r
SPM_ROWS = off_s
TOT_ROWS = off_h

NAT_CH = 16


def _sc_body(xT_hbm, t0, t1, t2, t3, t4, t5, t6, t7, t8, t9, t10, t11, t12,
             t13, t14, t15, out_hbm, x_v, idx_v, sub_v, w_v, rows_v,
             out_v, nat_v, pk_v, spm, pk_hbm, sem_a, sem_b):
    tables = [t0, t1, t2, t3, t4, t5, t6, t7, t8, t9, t10, t11, t12, t13,
              t14, t15]
    sems = [sem_a, sem_b]
    c = lax.axis_index("c")
    s = lax.axis_index("s")
    wid = s * NC + c
    lane = lax.iota(jnp.int32, LANES)
    sc_base = c * TOT_ROWS

    for l in range(N_LEVELS):
        sh = SHARE[l]
        ch = min(NAT_CH, sh)
        n_ch = sh // ch
        ng = ch * 8

        def rep_chunk(j, _, l=l, sh=sh, ch=ch, ng=ng):
            b0 = s * sh + j * ch
            pltpu.sync_copy(tables[l].at[pl.ds(b0, ch), :, :],
                            nat_v.at[pl.ds(0, ch), :, :])

            def rep_grp(g, _):
                e0 = g * LANES
                blk = lax.div(e0, BLK)
                eo = lax.rem(e0, BLK)
                f0 = nat_v[blk, 0, pl.ds(eo, LANES)]
                f1 = nat_v[blk, 1, pl.ds(eo, LANES)]
                pos = 2 * (e0 + lane)
                r = lax.shift_right_logical(pos, 4)
                sub = pos & 15
                plsc.store_scatter(pk_v, [r, sub], f0)
                plsc.store_scatter(pk_v, [r, sub + 1], f1)
                return 0

            lax.fori_loop(0, ng, rep_grp, 0)
            if l < N_SPM:
                row0 = LEVEL_OFF[l] + (b0 * (BLK // 8))
                pltpu.sync_copy(pk_v.at[pl.ds(0, ch * (BLK // 8)), :],
                                spm.at[pl.ds(row0, ch * (BLK // 8)), :])
            else:
                row0 = sc_base + LEVEL_OFF[l] + (b0 * (BLK // 8))
                pltpu.sync_copy(pk_v.at[pl.ds(0, ch * (BLK // 8)), :],
                                pk_hbm.at[pl.ds(row0, ch * (BLK // 8)), :])
            return 0

        lax.fori_loop(0, n_ch, rep_chunk, 0)

    plsc.subcore_barrier()

    def chunk_body(ci, carry):
        base = wid * PTS_PER_W + ci * P

        def idx_level(l, b):
            res, hsize = LEVELS[l]
            resf = jnp.float32(res)
            pow2 = (hsize & (hsize - 1)) == 0
            if l < N_SPM:
                base_l = LEVEL_OFF[l]
            else:
                base_l = sc_base + LEVEL_OFF[l]

            def idx_body(g, _, hsize=hsize, pow2=pow2, resf=resf, b=b,
                         base_l=base_l):
                p0 = g * LANES
                xs0 = x_v[0, pl.ds(p0, LANES)] * resf
                xs1 = x_v[1, pl.ds(p0, LANES)] * resf
                xs2 = x_v[2, pl.ds(p0, LANES)] * resf
                i0 = xs0.astype(jnp.int32)
                i1 = xs1.astype(jnp.int32)
                i2 = xs2.astype(jnp.int32)
                f0 = xs0 - i0.astype(jnp.float32)
                f1 = xs1 - i1.astype(jnp.float32)
                f2 = xs2 - i2.astype(jnp.float32)
                u0a = i0.astype(jnp.uint32)
                u1a = i1.astype(jnp.uint32) * jnp.uint32(PRIME1)
                u2a = i2.astype(jnp.uint32) * jnp.uint32(PRIME2)
                u0b = u0a + jnp.uint32(1)
                u1b = u1a + jnp.uint32(PRIME1)
                u2b = u2a + jnp.uint32(PRIME2)
                g0 = 1.0 - f0
                g1 = 1.0 - f1
                g2 = 1.0 - f2
                for cidx in range(8):
                    hx = u0b if (cidx & 1) else u0a
                    hy = u1b if (cidx & 2) else u1a
                    hz = u2b if (cidx & 4) else u2a
                    h = hx ^ hy ^ hz
                    if pow2:
                        h = h & jnp.uint32(hsize - 1)
                    else:
                        h = h % jnp.uint32(hsize)
                    h = h.astype(jnp.int32)
                    idx_v[b, pl.ds(cidx * P + p0, LANES)] = (
                        base_l + lax.shift_right_logical(h, 3))
                    sub_v[b, pl.ds(cidx * P + p0, LANES)] = (h & 7) * N_FEATS
                    wx = f0 if (cidx & 1) else g0
                    wy = f1 if (cidx & 2) else g1
                    wz = f2 if (cidx & 4) else g2
                    w_v[b, pl.ds(cidx * P + p0, LANES)] = wx * wy * wz
                return 0

            lax.fori_loop(0, G, idx_body, 0)
            src = spm if l < N_SPM else pk_hbm
            return pltpu.async_copy(src.at[idx_v.at[b]],
                                    rows_v.at[b], sems[b])

        def acc_level(l, b):
            def acc_body(g, _, l=l, b=b):
                p0 = g * LANES
                pos = p0 + lane
                bvec = jnp.full((LANES,), b, jnp.int32)
                acc0 = jnp.zeros((LANES,), jnp.float32)
                acc1 = jnp.zeros((LANES,), jnp.float32)
                for cidx in range(8):
                    rpos = pos + cidx * P
                    sub = sub_v[b, pl.ds(cidx * P + p0, LANES)]
                    r0 = plsc.load_gather(rows_v, [bvec, rpos, sub])
                    r1 = plsc.load_gather(rows_v, [bvec, rpos, sub + 1])
                    w = w_v[b, pl.ds(cidx * P + p0, LANES)]
                    acc0 = acc0 + w * r0
                    acc1 = acc1 + w * r1
                plsc.store_scatter(
                    out_v, [pos, jnp.full((LANES,), 2 * l, jnp.int32)], acc0)
                plsc.store_scatter(
                    out_v, [pos, jnp.full((LANES,), 2 * l + 1, jnp.int32)],
                    acc1)
                return 0

            lax.fori_loop(0, G, acc_body, 0)

        pltpu.sync_copy(xT_hbm.at[:, pl.ds(base, P)], x_v)
        cps = [None, None]
        cps[0] = idx_level(0, 0)
        for l in range(N_LEVELS):
            b = l % 2
            if l + 1 < N_LEVELS:
                cps[1 - b] = idx_level(l + 1, 1 - b)
            cps[b].wait()
            acc_level(l, b)
        pltpu.sync_copy(out_v, out_hbm.at[pl.ds(base, P), :])
        return carry

    lax.fori_loop(0, CHUNKS, chunk_body, 0)


def _native_view(table, hpad):
    hsize = table.shape[0]
    if hpad != hsize:
        table = jnp.concatenate(
            [table, jnp.zeros((hpad - hsize, N_FEATS), table.dtype)], axis=0)
    return table.reshape(hpad // BLK, BLK, N_FEATS).transpose(0, 2, 1)


def kernel(x, table_0, table_1, table_2, table_3, table_4, table_5, table_6,
           table_7, table_8, table_9, table_10, table_11, table_12, table_13,
           table_14, table_15):
    xT = x.T
    tables = [table_0, table_1, table_2, table_3, table_4, table_5, table_6,
              table_7, table_8, table_9, table_10, table_11, table_12,
              table_13, table_14, table_15]
    tviews = [_native_view(t, hp) for t, hp in zip(tables, PAD_H)]
    mesh = plsc.VectorSubcoreMesh(core_axis_name="c", subcore_axis_name="s")
    f = pl.kernel(
        _sc_body,
        out_type=jax.ShapeDtypeStruct((N_POINTS, OUT_F), jnp.float32),
        mesh=mesh,
        compiler_params=pltpu.CompilerParams(
            needs_layout_passes=False, use_tc_tiling_on_sc=False),
        scratch_types=[
            pltpu.VMEM((3, P), jnp.float32),
            pltpu.VMEM((2, K), jnp.int32),
            pltpu.VMEM((2, K), jnp.int32),
            pltpu.VMEM((2, K), jnp.float32),
            pltpu.VMEM((2, K, ROW), jnp.float32),
            pltpu.VMEM((P, OUT_F), jnp.float32),
            pltpu.VMEM((NAT_CH, N_FEATS, BLK), jnp.float32),
            pltpu.VMEM((NAT_CH * (BLK // 8), ROW), jnp.float32),
            pltpu.VMEM_SHARED((SPM_ROWS, ROW), jnp.float32),
            pltpu.HBM((NC * TOT_ROWS, ROW), jnp.float32),
            pltpu.SemaphoreType.DMA,
            pltpu.SemaphoreType.DMA,
        ],
    )
    return f(xT, *tviews)


# speedup vs baseline: 200.7924x; 1.0044x over previous
import math

import jax
import jax.numpy as jnp
from jax import lax
from jax.experimental import pallas as pl
from jax.experimental.pallas import tpu as pltpu
from jax.experimental.pallas import tpu_sc as plsc

DIM = 3
N_LEVELS = 16
N_FEATS = 2
LOG2_HASHMAP = 19
BASE_RES = 16
FINEST_RES = 512
N_POINTS = 524288
PRIME1 = 2654435761
PRIME2 = 805459861
OUT_F = 2 * N_LEVELS
ROW = 16
BLK = 128


def _levels():
    b = math.exp((math.log(FINEST_RES) - math.log(BASE_RES)) / (N_LEVELS - 1))
    out = []
    for i in range(N_LEVELS):
        res = math.floor(BASE_RES * (b ** i))
        hsize = min(res ** DIM, 2 ** LOG2_HASHMAP)
        out.append((res, hsize))
    return out


LEVELS = _levels()

NC = 2
NS = 16
LANES = 16
NW = NC * NS
PTS_PER_W = N_POINTS // NW
P = 256
CHUNKS = PTS_PER_W // P
G = P // LANES
K = 8 * P

PAD_H = [h if h % 2048 == 0 else max(2048, 1 << (h - 1).bit_length())
         for _, h in LEVELS]
BLOCKS = [h // BLK for h in PAD_H]
SHARE = [b // NS for b in BLOCKS]
ROWS_L = [h // 8 for h in PAD_H]
N_SPM = 6
LEVEL_OFF = []
off_s, off_h = 0, 0
for l, r in enumerate(ROWS_L):
    if l < N_SPM:
        LEVEL_OFF.append(off_s)
        off_s += r
    else:
        LEVEL_OFF.append(off_h)
        off_h += r
SPM_ROWS = off_s
TOT_ROWS = off_h

NAT_CH = 16


def _sc_body(xT_hbm, t0, t1, t2, t3, t4, t5, t6, t7, t8, t9, t10, t11, t12,
             t13, t14, t15, out_hbm, x_v, idx_v, sub_v, w_v, rows_v,
             out_v, nat_v, pk_v, spm, pk_hbm, sem_a, sem_b):
    tables = [t0, t1, t2, t3, t4, t5, t6, t7, t8, t9, t10, t11, t12, t13,
              t14, t15]
    sems = [sem_a, sem_b]
    c = lax.axis_index("c")
    s = lax.axis_index("s")
    wid = s * NC + c
    lane = lax.iota(jnp.int32, LANES)
    sc_base = c * TOT_ROWS

    for l in range(N_LEVELS):
        sh = SHARE[l]
        ch = max(d for d in range(1, min(NAT_CH, sh) + 1) if sh % d == 0)
        n_ch = sh // ch
        ng = ch * 8

        def rep_chunk(j, _, l=l, sh=sh, ch=ch, ng=ng):
            b0 = s * sh + j * ch
            pltpu.sync_copy(tables[l].at[pl.ds(b0, ch), :, :],
                            nat_v.at[pl.ds(0, ch), :, :])

            def rep_grp(g, _):
                e0 = g * LANES
                blk = lax.div(e0, BLK)
                eo = lax.rem(e0, BLK)
                f0 = nat_v[blk, 0, pl.ds(eo, LANES)]
                f1 = nat_v[blk, 1, pl.ds(eo, LANES)]
                pos = 2 * (e0 + lane)
                r = lax.shift_right_logical(pos, 4)
                sub = pos & 15
                plsc.store_scatter(pk_v, [r, sub], f0)
                plsc.store_scatter(pk_v, [r, sub + 1], f1)
                return 0

            lax.fori_loop(0, ng, rep_grp, 0)
            if l < N_SPM:
                row0 = LEVEL_OFF[l] + (b0 * (BLK // 8))
                pltpu.sync_copy(pk_v.at[pl.ds(0, ch * (BLK // 8)), :],
                                spm.at[pl.ds(row0, ch * (BLK // 8)), :])
            else:
                row0 = sc_base + LEVEL_OFF[l] + (b0 * (BLK // 8))
                pltpu.sync_copy(pk_v.at[pl.ds(0, ch * (BLK // 8)), :],
                                pk_hbm.at[pl.ds(row0, ch * (BLK // 8)), :])
            return 0

        lax.fori_loop(0, n_ch, rep_chunk, 0)

    plsc.subcore_barrier()

    def chunk_body(ci, carry):
        base = wid * PTS_PER_W + ci * P

        def idx_level(l, b):
            res, hsize = LEVELS[l]
            resf = jnp.float32(res)
            pow2 = (hsize & (hsize - 1)) == 0
            if l < N_SPM:
                base_l = LEVEL_OFF[l]
            else:
                base_l = sc_base + LEVEL_OFF[l]

            def idx_body(g, _, hsize=hsize, pow2=pow2, resf=resf, b=b,
                         base_l=base_l):
                p0 = g * LANES
                xs0 = x_v[0, pl.ds(p0, LANES)] * resf
                xs1 = x_v[1, pl.ds(p0, LANES)] * resf
                xs2 = x_v[2, pl.ds(p0, LANES)] * resf
                i0 = xs0.astype(jnp.int32)
                i1 = xs1.astype(jnp.int32)
                i2 = xs2.astype(jnp.int32)
                f0 = xs0 - i0.astype(jnp.float32)
                f1 = xs1 - i1.astype(jnp.float32)
                f2 = xs2 - i2.astype(jnp.float32)
                u0a = i0.astype(jnp.uint32)
                u1a = i1.astype(jnp.uint32) * jnp.uint32(PRIME1)
                u2a = i2.astype(jnp.uint32) * jnp.uint32(PRIME2)
                u0b = u0a + jnp.uint32(1)
                u1b = u1a + jnp.uint32(PRIME1)
                u2b = u2a + jnp.uint32(PRIME2)
                g0 = 1.0 - f0
                g1 = 1.0 - f1
                g2 = 1.0 - f2
                wyz = [g1 * g2, f1 * g2, g1 * f2, f1 * f2]
                for cidx in range(8):
                    hx = u0b if (cidx & 1) else u0a
                    hy = u1b if (cidx & 2) else u1a
                    hz = u2b if (cidx & 4) else u2a
                    h = hx ^ hy ^ hz
                    if pow2:
                        h = h & jnp.uint32(hsize - 1)
                    else:
                        h = h % jnp.uint32(hsize)
                    h = h.astype(jnp.int32)
                    idx_v[b, pl.ds(cidx * P + p0, LANES)] = (
                        base_l + lax.shift_right_logical(h, 3))
                    sub_v[b, pl.ds(cidx * P + p0, LANES)] = (h & 7) * N_FEATS
                    wx = f0 if (cidx & 1) else g0
                    w_v[b, pl.ds(cidx * P + p0, LANES)] = wx * wyz[cidx >> 1]
                return 0

            lax.fori_loop(0, G, idx_body, 0)
            src = spm if l < N_SPM else pk_hbm
            return pltpu.async_copy(src.at[idx_v.at[b]],
                                    rows_v.at[b], sems[b])

        def acc_level(l, b):
            def acc_body(g, _, l=l, b=b):
                p0 = g * LANES
                pos = p0 + lane
                bvec = jnp.full((LANES,), b, jnp.int32)
                acc0 = jnp.zeros((LANES,), jnp.float32)
                acc1 = jnp.zeros((LANES,), jnp.float32)
                for cidx in range(8):
                    rpos = pos + cidx * P
                    sub = sub_v[b, pl.ds(cidx * P + p0, LANES)]
                    r0 = plsc.load_gather(rows_v, [bvec, rpos, sub])
                    r1 = plsc.load_gather(rows_v, [bvec, rpos, sub + 1])
                    w = w_v[b, pl.ds(cidx * P + p0, LANES)]
                    acc0 = acc0 + w * r0
                    acc1 = acc1 + w * r1
                w0 = pos * OUT_F + (2 * l)
                plsc.store_scatter(
                    out_v, [lax.shift_right_logical(w0, 7), w0 & 127], acc0)
                w1 = w0 + 1
                plsc.store_scatter(
                    out_v, [lax.shift_right_logical(w1, 7), w1 & 127], acc1)
                return 0

            lax.fori_loop(0, G, acc_body, 0)

        pltpu.sync_copy(xT_hbm.at[:, pl.ds(base, P)], x_v)
        cps = [None, None]
        cps[0] = idx_level(0, 0)
        for l in range(N_LEVELS):
            b = l % 2
            if l + 1 < N_LEVELS:
                cps[1 - b] = idx_level(l + 1, 1 - b)
            cps[b].wait()
            acc_level(l, b)
        pltpu.sync_copy(out_v,
                        out_hbm.at[pl.ds(base * OUT_F // 128, P * OUT_F // 128), :])
        return carry

    lax.fori_loop(0, CHUNKS, chunk_body, 0)


def _native_view(table, hpad):
    hsize = table.shape[0]
    if hpad != hsize:
        table = jnp.concatenate(
            [table, jnp.zeros((hpad - hsize, N_FEATS), table.dtype)], axis=0)
    return table.reshape(hpad // BLK, BLK, N_FEATS).transpose(0, 2, 1)


def kernel(x, table_0, table_1, table_2, table_3, table_4, table_5, table_6,
           table_7, table_8, table_9, table_10, table_11, table_12, table_13,
           table_14, table_15):
    xT = x.T
    tables = [table_0, table_1, table_2, table_3, table_4, table_5, table_6,
              table_7, table_8, table_9, table_10, table_11, table_12,
              table_13, table_14, table_15]
    tviews = [_native_view(t, hp) for t, hp in zip(tables, PAD_H)]
    mesh = plsc.VectorSubcoreMesh(core_axis_name="c", subcore_axis_name="s")
    f = pl.kernel(
        _sc_body,
        out_type=jax.ShapeDtypeStruct((N_POINTS * OUT_F // 128, 128),
                                      jnp.float32),
        mesh=mesh,
        compiler_params=pltpu.CompilerParams(
            needs_layout_passes=False, use_tc_tiling_on_sc=False),
        scratch_types=[
            pltpu.VMEM((3, P), jnp.float32),
            pltpu.VMEM((2, K), jnp.int32),
            pltpu.VMEM((2, K), jnp.int32),
            pltpu.VMEM((2, K), jnp.float32),
            pltpu.VMEM((2, K, ROW), jnp.float32),
            pltpu.VMEM((P * OUT_F // 128, 128), jnp.float32),
            pltpu.VMEM((NAT_CH, N_FEATS, BLK), jnp.float32),
            pltpu.VMEM((NAT_CH * (BLK // 8), ROW), jnp.float32),
            pltpu.VMEM_SHARED((SPM_ROWS, ROW), jnp.float32),
            pltpu.HBM((NC * TOT_ROWS, ROW), jnp.float32),
            pltpu.SemaphoreType.DMA,
            pltpu.SemaphoreType.DMA,
        ],
    )
    return f(xT, *tviews).reshape(N_POINTS, OUT_F)


# speedup vs baseline: 229.5065x; 1.1430x over previous
import math

import jax
import jax.numpy as jnp
from jax import lax
from jax.experimental import pallas as pl
from jax.experimental.pallas import tpu as pltpu
from jax.experimental.pallas import tpu_sc as plsc

DIM = 3
N_LEVELS = 16
N_FEATS = 2
LOG2_HASHMAP = 19
BASE_RES = 16
FINEST_RES = 512
N_POINTS = 524288
PRIME1 = 2654435761
PRIME2 = 805459861
OUT_F = 2 * N_LEVELS
ROW = 16
BLK = 128


def _levels():
    b = math.exp((math.log(FINEST_RES) - math.log(BASE_RES)) / (N_LEVELS - 1))
    out = []
    for i in range(N_LEVELS):
        res = math.floor(BASE_RES * (b ** i))
        hsize = min(res ** DIM, 2 ** LOG2_HASHMAP)
        out.append((res, hsize))
    return out


LEVELS = _levels()

NC = 2
NS = 16
LANES = 16
NW = NC * NS
PTS_PER_W = N_POINTS // NW
P = 256
CHUNKS = PTS_PER_W // P
G = P // LANES
K = 8 * P

PAD_H = [h if h % 2048 == 0 else max(2048, 1 << (h - 1).bit_length())
         for _, h in LEVELS]
BLOCKS = [h // BLK for h in PAD_H]
SHARE = [b // NS for b in BLOCKS]
ROWS_L = [h // 8 for h in PAD_H]
N_SPM = 6
LEVEL_OFF = []
off_s, off_h = 0, 0
for l, r in enumerate(ROWS_L):
    if l < N_SPM:
        LEVEL_OFF.append(off_s)
        off_s += r
    else:
        LEVEL_OFF.append(off_h)
        off_h += r
SPM_ROWS = off_s
TOT_ROWS = off_h

NAT_CH = 16


def _sc_body(xT_hbm, t0, t1, t2, t3, t4, t5, t6, t7, t8, t9, t10, t11, t12,
             t13, t14, t15, out_hbm, x_v, idx_v, sub_v, w_v, rows_v,
             out_v, nat_v, pk_v, spm, pk_hbm, sem_a, sem_b):
    tables = [t0, t1, t2, t3, t4, t5, t6, t7, t8, t9, t10, t11, t12, t13,
              t14, t15]
    sems = [sem_a, sem_b]
    c = lax.axis_index("c")
    s = lax.axis_index("s")
    wid = s * NC + c
    lane = lax.iota(jnp.int32, LANES)
    sc_base = c * TOT_ROWS

    for l in range(N_LEVELS):
        sh = SHARE[l]
        ch = max(d for d in range(1, min(NAT_CH, sh) + 1) if sh % d == 0)
        n_ch = sh // ch
        ng = ch * 8

        def rep_chunk(j, _, l=l, sh=sh, ch=ch, ng=ng):
            b0 = s * sh + j * ch
            pltpu.sync_copy(tables[l].at[pl.ds(b0, ch), :, :],
                            nat_v.at[pl.ds(0, ch), :, :])

            def rep_grp(g, _):
                e0 = g * LANES
                blk = lax.div(e0, BLK)
                eo = lax.rem(e0, BLK)
                f0 = nat_v[blk, 0, pl.ds(eo, LANES)]
                f1 = nat_v[blk, 1, pl.ds(eo, LANES)]
                pos = 2 * (e0 + lane)
                r = lax.shift_right_logical(pos, 4)
                sub = pos & 15
                plsc.store_scatter(pk_v, [r, sub], f0)
                plsc.store_scatter(pk_v, [r, sub + 1], f1)
                return 0

            lax.fori_loop(0, ng, rep_grp, 0)
            if l < N_SPM:
                row0 = LEVEL_OFF[l] + (b0 * (BLK // 8))
                pltpu.sync_copy(pk_v.at[pl.ds(0, ch * (BLK // 8)), :],
                                spm.at[pl.ds(row0, ch * (BLK // 8)), :])
            else:
                row0 = sc_base + LEVEL_OFF[l] + (b0 * (BLK // 8))
                pltpu.sync_copy(pk_v.at[pl.ds(0, ch * (BLK // 8)), :],
                                pk_hbm.at[pl.ds(row0, ch * (BLK // 8)), :])
            return 0

        lax.fori_loop(0, n_ch, rep_chunk, 0)

    plsc.subcore_barrier()

    def chunk_body(ci, carry):
        base = wid * PTS_PER_W + ci * P

        def idx_level(l, b):
            res, hsize = LEVELS[l]
            resf = jnp.float32(res)
            pow2 = (hsize & (hsize - 1)) == 0
            if l < N_SPM:
                base_l = LEVEL_OFF[l]
            else:
                base_l = sc_base + LEVEL_OFF[l]

            def idx_body(g, _, hsize=hsize, pow2=pow2, resf=resf, b=b,
                         base_l=base_l):
                p0 = g * LANES
                xs0 = x_v[0, pl.ds(p0, LANES)] * resf
                xs1 = x_v[1, pl.ds(p0, LANES)] * resf
                xs2 = x_v[2, pl.ds(p0, LANES)] * resf
                i0 = xs0.astype(jnp.int32)
                i1 = xs1.astype(jnp.int32)
                i2 = xs2.astype(jnp.int32)
                f0 = xs0 - i0.astype(jnp.float32)
                f1 = xs1 - i1.astype(jnp.float32)
                f2 = xs2 - i2.astype(jnp.float32)
                u0a = i0.astype(jnp.uint32)
                u1a = i1.astype(jnp.uint32) * jnp.uint32(PRIME1)
                u2a = i2.astype(jnp.uint32) * jnp.uint32(PRIME2)
                u0b = u0a + jnp.uint32(1)
                u1b = u1a + jnp.uint32(PRIME1)
                u2b = u2a + jnp.uint32(PRIME2)
                g0 = 1.0 - f0
                g1 = 1.0 - f1
                g2 = 1.0 - f2
                wyz = [g1 * g2, f1 * g2, g1 * f2, f1 * f2]
                for cidx in range(8):
                    hx = u0b if (cidx & 1) else u0a
                    hy = u1b if (cidx & 2) else u1a
                    hz = u2b if (cidx & 4) else u2a
                    h = hx ^ hy ^ hz
                    if pow2:
                        h = h & jnp.uint32(hsize - 1)
                    else:
                        h = h % jnp.uint32(hsize)
                    h = h.astype(jnp.int32)
                    idx_v[b, pl.ds(cidx * P + p0, LANES)] = (
                        base_l + lax.shift_right_logical(h, 3))
                    sub_v[b, pl.ds(cidx * P + p0, LANES)] = (h & 7) * N_FEATS
                    wx = f0 if (cidx & 1) else g0
                    w_v[b, pl.ds(cidx * P + p0, LANES)] = wx * wyz[cidx >> 1]
                return 0

            lax.fori_loop(0, G, idx_body, 0)
            src = spm if l < N_SPM else pk_hbm
            return pltpu.async_copy(src.at[idx_v.at[b]],
                                    rows_v.at[b], sems[b])

        def acc_level(l, b):
            def acc_body(g, _, l=l, b=b):
                p0 = g * LANES
                pos = p0 + lane
                bvec = jnp.full((LANES,), b, jnp.int32)
                acc0 = jnp.zeros((LANES,), jnp.float32)
                acc1 = jnp.zeros((LANES,), jnp.float32)
                for cidx in range(8):
                    rpos = pos + cidx * P
                    sub = sub_v[b, pl.ds(cidx * P + p0, LANES)]
                    r0 = plsc.load_gather(rows_v, [bvec, rpos, sub])
                    r1 = plsc.load_gather(rows_v, [bvec, rpos, sub + 1])
                    w = w_v[b, pl.ds(cidx * P + p0, LANES)]
                    acc0 = acc0 + w * r0
                    acc1 = acc1 + w * r1
                t0v = lax.shift_right_logical(pos, 7)
                pov = pos & 127
                f0c, f1c = 2 * l, 2 * l + 1
                plsc.store_scatter(
                    out_v, [jnp.full((LANES,), f0c >> 3, jnp.int32), t0v,
                            jnp.full((LANES,), f0c & 7, jnp.int32), pov],
                    acc0)
                plsc.store_scatter(
                    out_v, [jnp.full((LANES,), f1c >> 3, jnp.int32), t0v,
                            jnp.full((LANES,), f1c & 7, jnp.int32), pov],
                    acc1)
                return 0

            lax.fori_loop(0, G, acc_body, 0)

        pltpu.sync_copy(xT_hbm.at[:, pl.ds(base, P)], x_v)
        cps = [None, None]
        cps[0] = idx_level(0, 0)
        for l in range(N_LEVELS):
            b = l % 2
            if l + 1 < N_LEVELS:
                cps[1 - b] = idx_level(l + 1, 1 - b)
            cps[b].wait()
            acc_level(l, b)
        t0_0 = base // 128
        for t1 in range(OUT_F // 8):
            pltpu.sync_copy(out_v.at[t1],
                            out_hbm.at[t1, pl.ds(t0_0, P // 128), :, :])
        return carry

    lax.fori_loop(0, CHUNKS, chunk_body, 0)


def _native_view(table, hpad):
    hsize = table.shape[0]
    if hpad != hsize:
        table = jnp.concatenate(
            [table, jnp.zeros((hpad - hsize, N_FEATS), table.dtype)], axis=0)
    return table.reshape(hpad // BLK, BLK, N_FEATS).transpose(0, 2, 1)


def kernel(x, table_0, table_1, table_2, table_3, table_4, table_5, table_6,
           table_7, table_8, table_9, table_10, table_11, table_12, table_13,
           table_14, table_15):
    xT = x.T
    tables = [table_0, table_1, table_2, table_3, table_4, table_5, table_6,
              table_7, table_8, table_9, table_10, table_11, table_12,
              table_13, table_14, table_15]
    tviews = [_native_view(t, hp) for t, hp in zip(tables, PAD_H)]
    mesh = plsc.VectorSubcoreMesh(core_axis_name="c", subcore_axis_name="s")
    f = pl.kernel(
        _sc_body,
        out_type=jax.ShapeDtypeStruct(
            (OUT_F // 8, N_POINTS // 128, 8, 128), jnp.float32),
        mesh=mesh,
        compiler_params=pltpu.CompilerParams(
            needs_layout_passes=False, use_tc_tiling_on_sc=False),
        scratch_types=[
            pltpu.VMEM((3, P), jnp.float32),
            pltpu.VMEM((2, K), jnp.int32),
            pltpu.VMEM((2, K), jnp.int32),
            pltpu.VMEM((2, K), jnp.float32),
            pltpu.VMEM((2, K, ROW), jnp.float32),
            pltpu.VMEM((OUT_F // 8, P // 128, 8, 128), jnp.float32),
            pltpu.VMEM((NAT_CH, N_FEATS, BLK), jnp.float32),
            pltpu.VMEM((NAT_CH * (BLK // 8), ROW), jnp.float32),
            pltpu.VMEM_SHARED((SPM_ROWS, ROW), jnp.float32),
            pltpu.HBM((NC * TOT_ROWS, ROW), jnp.float32),
            pltpu.SemaphoreType.DMA,
            pltpu.SemaphoreType.DMA,
        ],
    )
    out4 = f(xT, *tviews)
    return out4.transpose(1, 3, 0, 2).reshape(N_POINTS, OUT_F)


# speedup vs baseline: 230.1169x; 1.0027x over previous
import math

import jax
import jax.numpy as jnp
from jax import lax
from jax.experimental import pallas as pl
from jax.experimental.pallas import tpu as pltpu
from jax.experimental.pallas import tpu_sc as plsc

DIM = 3
N_LEVELS = 16
N_FEATS = 2
LOG2_HASHMAP = 19
BASE_RES = 16
FINEST_RES = 512
N_POINTS = 524288
PRIME1 = 2654435761
PRIME2 = 805459861
OUT_F = 2 * N_LEVELS
ROW = 16
BLK = 128


def _levels():
    b = math.exp((math.log(FINEST_RES) - math.log(BASE_RES)) / (N_LEVELS - 1))
    out = []
    for i in range(N_LEVELS):
        res = math.floor(BASE_RES * (b ** i))
        hsize = min(res ** DIM, 2 ** LOG2_HASHMAP)
        out.append((res, hsize))
    return out


LEVELS = _levels()

NC = 2
NS = 16
LANES = 16
NW = NC * NS
PTS_PER_W = N_POINTS // NW
P = 256
CHUNKS = PTS_PER_W // P
G = P // LANES
K = 8 * P

PAD_H = [h if h % 2048 == 0 else max(2048, 1 << (h - 1).bit_length())
         for _, h in LEVELS]
BLOCKS = [h // BLK for h in PAD_H]
SHARE = [b // NS for b in BLOCKS]
ROWS_L = [h // 8 for h in PAD_H]
N_SPM = 6
LEVEL_OFF = []
off_s, off_h = 0, 0
for l, r in enumerate(ROWS_L):
    if l < N_SPM:
        LEVEL_OFF.append(off_s)
        off_s += r
    else:
        LEVEL_OFF.append(off_h)
        off_h += r
SPM_ROWS = off_s
TOT_ROWS = off_h

NAT_CH = 16


def _sc_body(xT_hbm, t0, t1, t2, t3, t4, t5, t6, t7, t8, t9, t10, t11, t12,
             t13, t14, t15, out_hbm, x_v, idx_v, sub_v, w_v, rows_v,
             out_v, nat_v, pk_v, spm, pk_hbm, sem_a, sem_b):
    tables = [t0, t1, t2, t3, t4, t5, t6, t7, t8, t9, t10, t11, t12, t13,
              t14, t15]
    sems = [sem_a, sem_b]
    c = lax.axis_index("c")
    s = lax.axis_index("s")
    wid = s * NC + c
    lane = lax.iota(jnp.int32, LANES)
    sc_base = c * TOT_ROWS

    for l in range(N_LEVELS):
        sh = SHARE[l]
        ch = max(d for d in range(1, min(NAT_CH, sh) + 1) if sh % d == 0)
        n_ch = sh // ch
        ng = ch * 8

        def rep_chunk(j, _, l=l, sh=sh, ch=ch, ng=ng):
            b0 = s * sh + j * ch
            pltpu.sync_copy(tables[l].at[pl.ds(b0, ch), :, :],
                            nat_v.at[pl.ds(0, ch), :, :])

            def rep_grp(g, _):
                e0 = g * LANES
                blk = lax.div(e0, BLK)
                eo = lax.rem(e0, BLK)
                f0 = nat_v[blk, 0, pl.ds(eo, LANES)]
                f1 = nat_v[blk, 1, pl.ds(eo, LANES)]
                pos = 2 * (e0 + lane)
                r = lax.shift_right_logical(pos, 4)
                sub = pos & 15
                plsc.store_scatter(pk_v, [r, sub], f0)
                plsc.store_scatter(pk_v, [r, sub + 1], f1)
                return 0

            lax.fori_loop(0, ng, rep_grp, 0)
            if l < N_SPM:
                row0 = LEVEL_OFF[l] + (b0 * (BLK // 8))
                pltpu.sync_copy(pk_v.at[pl.ds(0, ch * (BLK // 8)), :],
                                spm.at[pl.ds(row0, ch * (BLK // 8)), :])
            else:
                row0 = sc_base + LEVEL_OFF[l] + (b0 * (BLK // 8))
                pltpu.sync_copy(pk_v.at[pl.ds(0, ch * (BLK // 8)), :],
                                pk_hbm.at[pl.ds(row0, ch * (BLK // 8)), :])
            return 0

        lax.fori_loop(0, n_ch, rep_chunk, 0)

    plsc.subcore_barrier()

    def chunk_body(ci, carry):
        base = wid * PTS_PER_W + ci * P

        def idx_level(l, b):
            res, hsize = LEVELS[l]
            resf = jnp.float32(res)
            pow2 = (hsize & (hsize - 1)) == 0
            if l < N_SPM:
                base_l = LEVEL_OFF[l]
            else:
                base_l = sc_base + LEVEL_OFF[l]

            def idx_one(p0, hsize=hsize, pow2=pow2, resf=resf, b=b,
                        base_l=base_l):
                xs0 = x_v[0, pl.ds(p0, LANES)] * resf
                xs1 = x_v[1, pl.ds(p0, LANES)] * resf
                xs2 = x_v[2, pl.ds(p0, LANES)] * resf
                i0 = xs0.astype(jnp.int32)
                i1 = xs1.astype(jnp.int32)
                i2 = xs2.astype(jnp.int32)
                f0 = xs0 - i0.astype(jnp.float32)
                f1 = xs1 - i1.astype(jnp.float32)
                f2 = xs2 - i2.astype(jnp.float32)
                u0a = i0.astype(jnp.uint32)
                u1a = i1.astype(jnp.uint32) * jnp.uint32(PRIME1)
                u2a = i2.astype(jnp.uint32) * jnp.uint32(PRIME2)
                u0b = u0a + jnp.uint32(1)
                u1b = u1a + jnp.uint32(PRIME1)
                u2b = u2a + jnp.uint32(PRIME2)
                g0 = 1.0 - f0
                g1 = 1.0 - f1
                g2 = 1.0 - f2
                wyz = [g1 * g2, f1 * g2, g1 * f2, f1 * f2]
                for cidx in range(8):
                    hx = u0b if (cidx & 1) else u0a
                    hy = u1b if (cidx & 2) else u1a
                    hz = u2b if (cidx & 4) else u2a
                    h = hx ^ hy ^ hz
                    if pow2:
                        h = h & jnp.uint32(hsize - 1)
                    else:
                        h = h % jnp.uint32(hsize)
                    h = h.astype(jnp.int32)
                    idx_v[b, pl.ds(cidx * P + p0, LANES)] = (
                        base_l + lax.shift_right_logical(h, 3))
                    sub_v[b, pl.ds(cidx * P + p0, LANES)] = (h & 7) * N_FEATS
                    wx = f0 if (cidx & 1) else g0
                    w_v[b, pl.ds(cidx * P + p0, LANES)] = wx * wyz[cidx >> 1]

            def idx_body(g, _):
                idx_one(g * (2 * LANES))
                idx_one(g * (2 * LANES) + LANES)
                return 0

            lax.fori_loop(0, G // 2, idx_body, 0)
            src = spm if l < N_SPM else pk_hbm
            return pltpu.async_copy(src.at[idx_v.at[b]],
                                    rows_v.at[b], sems[b])

        def acc_level(l, b):
            def acc_one(p0, l=l, b=b):
                pos = p0 + lane
                bvec = jnp.full((LANES,), b, jnp.int32)
                acc0 = jnp.zeros((LANES,), jnp.float32)
                acc1 = jnp.zeros((LANES,), jnp.float32)
                for cidx in range(8):
                    rpos = pos + cidx * P
                    sub = sub_v[b, pl.ds(cidx * P + p0, LANES)]
                    r0 = plsc.load_gather(rows_v, [bvec, rpos, sub])
                    r1 = plsc.load_gather(rows_v, [bvec, rpos, sub + 1])
                    w = w_v[b, pl.ds(cidx * P + p0, LANES)]
                    acc0 = acc0 + w * r0
                    acc1 = acc1 + w * r1
                t0v = lax.shift_right_logical(pos, 7)
                pov = pos & 127
                f0c, f1c = 2 * l, 2 * l + 1
                plsc.store_scatter(
                    out_v, [jnp.full((LANES,), f0c >> 3, jnp.int32), t0v,
                            jnp.full((LANES,), f0c & 7, jnp.int32), pov],
                    acc0)
                plsc.store_scatter(
                    out_v, [jnp.full((LANES,), f1c >> 3, jnp.int32), t0v,
                            jnp.full((LANES,), f1c & 7, jnp.int32), pov],
                    acc1)

            def acc_body(g, _):
                acc_one(g * (2 * LANES))
                acc_one(g * (2 * LANES) + LANES)
                return 0

            lax.fori_loop(0, G // 2, acc_body, 0)

        pltpu.sync_copy(xT_hbm.at[:, pl.ds(base, P)], x_v)
        cps = [None, None]
        cps[0] = idx_level(0, 0)
        for l in range(N_LEVELS):
            b = l % 2
            if l + 1 < N_LEVELS:
                cps[1 - b] = idx_level(l + 1, 1 - b)
            cps[b].wait()
            acc_level(l, b)
        t0_0 = base // 128
        for t1 in range(OUT_F // 8):
            pltpu.sync_copy(out_v.at[t1],
                            out_hbm.at[t1, pl.ds(t0_0, P // 128), :, :])
        return carry

    lax.fori_loop(0, CHUNKS, chunk_body, 0)


def _native_view(table, hpad):
    hsize = table.shape[0]
    if hpad != hsize:
        table = jnp.concatenate(
            [table, jnp.zeros((hpad - hsize, N_FEATS), table.dtype)], axis=0)
    return table.reshape(hpad // BLK, BLK, N_FEATS).transpose(0, 2, 1)


def kernel(x, table_0, table_1, table_2, table_3, table_4, table_5, table_6,
           table_7, table_8, table_9, table_10, table_11, table_12, table_13,
           table_14, table_15):
    xT = x.T
    tables = [table_0, table_1, table_2, table_3, table_4, table_5, table_6,
              table_7, table_8, table_9, table_10, table_11, table_12,
              table_13, table_14, table_15]
    tviews = [_native_view(t, hp) for t, hp in zip(tables, PAD_H)]
    mesh = plsc.VectorSubcoreMesh(core_axis_name="c", subcore_axis_name="s")
    f = pl.kernel(
        _sc_body,
        out_type=jax.ShapeDtypeStruct(
            (OUT_F // 8, N_POINTS // 128, 8, 128), jnp.float32),
        mesh=mesh,
        compiler_params=pltpu.CompilerParams(
            needs_layout_passes=False, use_tc_tiling_on_sc=False),
        scratch_types=[
            pltpu.VMEM((3, P), jnp.float32),
            pltpu.VMEM((2, K), jnp.int32),
            pltpu.VMEM((2, K), jnp.int32),
            pltpu.VMEM((2, K), jnp.float32),
            pltpu.VMEM((2, K, ROW), jnp.float32),
            pltpu.VMEM((OUT_F // 8, P // 128, 8, 128), jnp.float32),
            pltpu.VMEM((NAT_CH, N_FEATS, BLK), jnp.float32),
            pltpu.VMEM((NAT_CH * (BLK // 8), ROW), jnp.float32),
            pltpu.VMEM_SHARED((SPM_ROWS, ROW), jnp.float32),
            pltpu.HBM((NC * TOT_ROWS, ROW), jnp.float32),
            pltpu.SemaphoreType.DMA,
            pltpu.SemaphoreType.DMA,
        ],
    )
    out4 = f(xT, *tviews)
    return out4.transpose(1, 3, 0, 2).reshape(N_POINTS, OUT_F)
